# baseline (speedup 1.0000x reference)
import dataclasses
import functools

import jax
import jax.numpy as jnp
from jax import lax
from jax.experimental import pallas as pl
from jax.experimental.pallas import tpu as pltpu
from jax.experimental.pallas import tpu_sc as plsc

VOCAB = 100000
MAX_LEN = 200
EMB = 100
BATCH = 4096

NUM_TILES = 32
ROWS_PER_TILE = BATCH // NUM_TILES
VCHUNK = 5000


SPLIT = 128
NB = MAX_LEN - SPLIT
VPAD = 100352
VCHUNKM = VPAD // 4


def _proj_body(tblt_ref, wq_ref, out_ref):
    dwr = wq_ref[0] - wq_ref[1]
    tblt = tblt_ref[...]
    a = lax.dot_general(tblt, dwr[0], (((0,), (1,)), ((), ())),
                        preferred_element_type=jnp.float32)
    b2 = lax.dot_general(tblt, dwr[1], (((0,), (1,)), ((), ())),
                         preferred_element_type=jnp.float32)
    ai = lax.bitcast_convert_type(a, jnp.int32) + jnp.int32(0x8000)
    bi = lax.bitcast_convert_type(b2, jnp.int32) + jnp.int32(0x8000)
    lo = jnp.bitwise_and(lax.shift_right_logical(ai, 16), jnp.int32(0xFFFF))
    hi = jnp.bitwise_and(bi, jnp.int32(-65536))
    out_ref[...] = jnp.bitwise_or(hi, lo)


def _project(tableT, Wq):
    return pl.pallas_call(
        _proj_body,
        grid=(VPAD // VCHUNKM,),
        in_specs=[
            pl.BlockSpec((EMB, VCHUNKM), lambda i: (0, i)),
            pl.BlockSpec((2, 2, SPLIT, EMB), lambda i: (0, 0, 0, 0)),
        ],
        out_specs=pl.BlockSpec((VCHUNKM, SPLIT), lambda i: (i, 0)),
        out_shape=jax.ShapeDtypeStruct((VPAD, SPLIT), jnp.int32),
    )(tableT, Wq)


def _sc_gather_sum(dflat, inpT):
    mesh = plsc.VectorSubcoreMesh(core_axis_name="c", subcore_axis_name="s")
    n_per_tile = MAX_LEN * ROWS_PER_TILE
    nseg = ROWS_PER_TILE // 16
    cp = pltpu.CompilerParams()
    if "needs_layout_passes" in pltpu.CompilerParams.__dataclass_fields__:
        cp = dataclasses.replace(cp, needs_layout_passes=False)
    CH = 25
    NCH = MAX_LEN // CH
    CHN = CH * ROWS_PER_TILE

    @functools.partial(
        pl.kernel,
        out_type=jax.ShapeDtypeStruct((BATCH,), jnp.float32),
        mesh=mesh,
        compiler_params=cp,
        scratch_types=[
            pltpu.VMEM((MAX_LEN, ROWS_PER_TILE), jnp.int32),
            pltpu.VMEM((n_per_tile,), jnp.int32),
            pltpu.VMEM((n_per_tile,), jnp.int32),
            pltpu.VMEM((ROWS_PER_TILE,), jnp.float32),
            pltpu.SemaphoreType.DMA,
        ],
    )
    def kern(dflat_hbm, inpt_hbm, out_hbm, inpt_v, idx_v, vals_v, dvec_v, sem):
        wid = lax.axis_index("s") * 2 + lax.axis_index("c")
        pltpu.sync_copy(
            inpt_hbm.at[:, pl.ds(wid * ROWS_PER_TILE, ROWS_PER_TILE)], inpt_v)

        def body_idx(p, _):
            pm = jnp.where(p < SPLIT, p, p - SPLIT)
            base = p * ROWS_PER_TILE
            for k in range(nseg):
                idx_v[pl.ds(base + 16 * k, 16)] = (
                    inpt_v[p, pl.ds(16 * k, 16)] * SPLIT + pm)
            return 0

        def body_lo(p, acc):
            base = p * ROWS_PER_TILE
            return tuple(
                acc[k] + plsc.bitcast(
                    lax.shift_left(vals_v[pl.ds(base + 16 * k, 16)], 16),
                    jnp.float32)
                for k in range(nseg))

        def body_hi(p, acc):
            base = p * ROWS_PER_TILE
            return tuple(
                acc[k] + plsc.bitcast(
                    jnp.bitwise_and(vals_v[pl.ds(base + 16 * k, 16)],
                                    jnp.int32(-65536)),
                    jnp.float32)
                for k in range(nseg))

        def fire(c):
            return pltpu.async_copy(
                dflat_hbm.at[idx_v.at[pl.ds(c * CHN, CHN)]],
                vals_v.at[pl.ds(c * CHN, CHN)], sem)

        def reduce_chunk(c, acc):
            p0, p1 = c * CH, (c + 1) * CH
            if p0 < SPLIT:
                acc = lax.fori_loop(p0, min(p1, SPLIT), body_lo, acc)
            if p1 > SPLIT:
                acc = lax.fori_loop(max(p0, SPLIT), p1, body_hi, acc)
            return acc

        lax.fori_loop(0, CH, body_idx, 0)
        prev = fire(0)
        acc = tuple(jnp.zeros((16,), jnp.float32) for _ in range(nseg))
        for c in range(1, NCH):
            lax.fori_loop(c * CH, (c + 1) * CH, body_idx, 0)
            cur = fire(c)
            prev.wait()
            acc = reduce_chunk(c - 1, acc)
            prev = cur
        prev.wait()
        acc = reduce_chunk(NCH - 1, acc)

        for k in range(nseg):
            dvec_v[pl.ds(16 * k, 16)] = acc[k]
        pltpu.sync_copy(dvec_v, out_hbm.at[pl.ds(wid * ROWS_PER_TILE,
                                                 ROWS_PER_TILE)])

    return kern(dflat, inpT)


def _finish_body(d_ref, b_ref, o0_ref, o1_ref):
    dt = d_ref[...] + (b_ref[0] - b_ref[1])
    o0 = -(jnp.maximum(-dt, 0.0) + jnp.log1p(jnp.exp(-jnp.abs(dt))))
    o0_ref[...] = o0
    o1_ref[...] = o0 - dt


def _finish(dmat, b):
    return pl.pallas_call(
        _finish_body,
        in_specs=[
            pl.BlockSpec(dmat.shape, lambda: (0, 0)),
            pl.BlockSpec(memory_space=pltpu.SMEM),
        ],
        out_specs=[
            pl.BlockSpec(dmat.shape, lambda: (0, 0)),
            pl.BlockSpec(dmat.shape, lambda: (0, 0)),
        ],
        out_shape=[
            jax.ShapeDtypeStruct(dmat.shape, jnp.float32),
            jax.ShapeDtypeStruct(dmat.shape, jnp.float32),
        ],
    )(dmat, b)


def kernel(inp, table, W, b):
    tableT = jnp.swapaxes(table, 0, 1)
    inpT = jnp.swapaxes(inp, 0, 1)

    Wr = W.reshape(2, MAX_LEN, EMB)
    Wrp = jnp.pad(Wr, ((0, 0), (0, 2 * SPLIT - MAX_LEN), (0, 0)))
    Wq = Wrp.reshape(2, 2, SPLIT, EMB)
    packed = _project(tableT, Wq)

    d = _sc_gather_sum(packed.reshape(-1), inpT)
    o0, o1 = _finish(d.reshape(NUM_TILES, ROWS_PER_TILE), b)
    return jnp.stack([o0.reshape(-1), o1.reshape(-1)], axis=-1)


# speedup vs baseline: 1.0130x; 1.0130x over previous
import dataclasses
import functools

import jax
import jax.numpy as jnp
from jax import lax
from jax.experimental import pallas as pl
from jax.experimental.pallas import tpu as pltpu
from jax.experimental.pallas import tpu_sc as plsc

VOCAB = 100000
MAX_LEN = 200
EMB = 100
BATCH = 4096

NUM_TILES = 32
ROWS_PER_TILE = BATCH // NUM_TILES
SPLIT = 128
VPAD = 100352
VCHUNKM = VPAD // 4


def _proj_body(tblt_ref, wq_ref, out_ref):
    dwr = wq_ref[0] - wq_ref[1]
    tblt = tblt_ref[...]
    a = lax.dot_general(tblt, dwr[0], (((0,), (1,)), ((), ())),
                        preferred_element_type=jnp.float32)
    b2 = lax.dot_general(tblt, dwr[1], (((0,), (1,)), ((), ())),
                         preferred_element_type=jnp.float32)
    ai = lax.bitcast_convert_type(a, jnp.int32) + jnp.int32(0x8000)
    bi = lax.bitcast_convert_type(b2, jnp.int32) + jnp.int32(0x8000)
    lo = jnp.bitwise_and(lax.shift_right_logical(ai, 16), jnp.int32(0xFFFF))
    hi = jnp.bitwise_and(bi, jnp.int32(-65536))
    out_ref[...] = jnp.bitwise_or(hi, lo)


def _project(tableT, Wq):
    return pl.pallas_call(
        _proj_body,
        grid=(VPAD // VCHUNKM,),
        in_specs=[
            pl.BlockSpec((EMB, VCHUNKM), lambda i: (0, i)),
            pl.BlockSpec((2, 2, SPLIT, EMB), lambda i: (0, 0, 0, 0)),
        ],
        out_specs=pl.BlockSpec((VCHUNKM, SPLIT), lambda i: (i, 0)),
        out_shape=jax.ShapeDtypeStruct((VPAD, SPLIT), jnp.int32),
    )(tableT, Wq)


def _sc_gather_sum(dflat, inpT):
    mesh = plsc.VectorSubcoreMesh(core_axis_name="c", subcore_axis_name="s")
    n_per_tile = MAX_LEN * ROWS_PER_TILE
    nseg = ROWS_PER_TILE // 16
    cp = pltpu.CompilerParams()
    if "needs_layout_passes" in pltpu.CompilerParams.__dataclass_fields__:
        cp = dataclasses.replace(cp, needs_layout_passes=False)
    CH = 50
    NCH = MAX_LEN // CH
    CHN = CH * ROWS_PER_TILE

    @functools.partial(
        pl.kernel,
        out_type=jax.ShapeDtypeStruct((BATCH,), jnp.float32),
        mesh=mesh,
        compiler_params=cp,
        scratch_types=[
            pltpu.VMEM((MAX_LEN, ROWS_PER_TILE), jnp.int32),
            pltpu.VMEM((n_per_tile,), jnp.int32),
            pltpu.VMEM((n_per_tile,), jnp.int32),
            pltpu.VMEM((ROWS_PER_TILE,), jnp.float32),
            pltpu.SemaphoreType.DMA,
        ],
    )
    def kern(dflat_hbm, inpt_hbm, out_hbm, inpt_v, idx_v, vals_v, dvec_v, sem):
        wid = lax.axis_index("s") * 2 + lax.axis_index("c")
        pltpu.sync_copy(
            inpt_hbm.at[:, pl.ds(wid * ROWS_PER_TILE, ROWS_PER_TILE)], inpt_v)

        def body_idx(p, _):
            pm = jnp.where(p < SPLIT, p, p - SPLIT)
            base = p * ROWS_PER_TILE
            for k in range(nseg):
                idx_v[pl.ds(base + 16 * k, 16)] = (
                    inpt_v[p, pl.ds(16 * k, 16)] * SPLIT + pm)
            return 0

        def body_lo(p, acc):
            base = p * ROWS_PER_TILE
            return tuple(
                acc[k] + plsc.bitcast(
                    lax.shift_left(vals_v[pl.ds(base + 16 * k, 16)], 16),
                    jnp.float32)
                for k in range(nseg))

        def body_hi(p, acc):
            base = p * ROWS_PER_TILE
            return tuple(
                acc[k] + plsc.bitcast(
                    jnp.bitwise_and(vals_v[pl.ds(base + 16 * k, 16)],
                                    jnp.int32(-65536)),
                    jnp.float32)
                for k in range(nseg))

        def fire(c):
            return pltpu.async_copy(
                dflat_hbm.at[idx_v.at[pl.ds(c * CHN, CHN)]],
                vals_v.at[pl.ds(c * CHN, CHN)], sem)

        def reduce_chunk(c, acc):
            p0, p1 = c * CH, (c + 1) * CH
            if p0 < SPLIT:
                acc = lax.fori_loop(p0, min(p1, SPLIT), body_lo, acc)
            if p1 > SPLIT:
                acc = lax.fori_loop(max(p0, SPLIT), p1, body_hi, acc)
            return acc

        lax.fori_loop(0, CH, body_idx, 0)
        prev = fire(0)
        acc = tuple(jnp.zeros((16,), jnp.float32) for _ in range(nseg))
        for c in range(1, NCH):
            lax.fori_loop(c * CH, (c + 1) * CH, body_idx, 0)
            cur = fire(c)
            prev.wait()
            acc = reduce_chunk(c - 1, acc)
            prev = cur
        prev.wait()
        acc = reduce_chunk(NCH - 1, acc)

        for k in range(nseg):
            dvec_v[pl.ds(16 * k, 16)] = acc[k]
        pltpu.sync_copy(dvec_v, out_hbm.at[pl.ds(wid * ROWS_PER_TILE,
                                                 ROWS_PER_TILE)])

    return kern(dflat, inpT)


def _finish_body(d_ref, b_ref, o0_ref, o1_ref):
    dt = d_ref[...] + (b_ref[0] - b_ref[1])
    o0 = -(jnp.maximum(-dt, 0.0) + jnp.log1p(jnp.exp(-jnp.abs(dt))))
    o0_ref[...] = o0
    o1_ref[...] = o0 - dt


def _finish(dmat, b):
    return pl.pallas_call(
        _finish_body,
        in_specs=[
            pl.BlockSpec(dmat.shape, lambda: (0, 0)),
            pl.BlockSpec(memory_space=pltpu.SMEM),
        ],
        out_specs=[
            pl.BlockSpec(dmat.shape, lambda: (0, 0)),
            pl.BlockSpec(dmat.shape, lambda: (0, 0)),
        ],
        out_shape=[
            jax.ShapeDtypeStruct(dmat.shape, jnp.float32),
            jax.ShapeDtypeStruct(dmat.shape, jnp.float32),
        ],
    )(dmat, b)


def kernel(inp, table, W, b):
    tableT = jnp.swapaxes(table, 0, 1)
    inpT = jnp.swapaxes(inp, 0, 1)

    Wr = W.reshape(2, MAX_LEN, EMB)
    Wrp = jnp.pad(Wr, ((0, 0), (0, 2 * SPLIT - MAX_LEN), (0, 0)))
    Wq = Wrp.reshape(2, 2, SPLIT, EMB)
    packed = _project(tableT, Wq)

    d = _sc_gather_sum(packed.reshape(-1), inpT)
    o0, o1 = _finish(d.reshape(NUM_TILES, ROWS_PER_TILE), b)
    return jnp.stack([o0.reshape(-1), o1.reshape(-1)], axis=-1)
